# baseline (speedup 1.0000x reference)
import functools

import jax
import jax.numpy as jnp
from jax import lax
from jax.experimental import pallas as pl
from jax.experimental.pallas import tpu as pltpu
from jax.experimental.pallas import tpu_sc as plsc

_MAX_LEN = 8192
_CHUNK = 16
_NBUF = 4
_OG = 3


@functools.lru_cache(maxsize=None)
def _make_gather(B, V, D):
    info = plsc.get_sparse_core_info()
    nc, ns = info.num_cores, info.num_subcores
    nw = nc * ns
    b_per_w = B // nw
    n_chunks = b_per_w // _CHUNK
    assert b_per_w * nw == B and n_chunks * _CHUNK == b_per_w
    assert n_chunks % _NBUF == 0 and n_chunks >= 2 * _NBUF

    mesh = plsc.VectorSubcoreMesh(core_axis_name="c", subcore_axis_name="s")

    @functools.partial(
        pl.kernel,
        mesh=mesh,
        out_type=jax.ShapeDtypeStruct((B, D), jnp.float32),
        scratch_types=[
            pltpu.VMEM((b_per_w,), jnp.int32),
            pltpu.VMEM((_NBUF, _CHUNK, D), jnp.float32),
        ]
        + [pltpu.SemaphoreType.DMA] * (2 * _NBUF),
    )
    def gather_kernel(idx_hbm, table_hbm, out_hbm, idx_v, rows_v, *sems):
        gsem, wsem = sems[:_NBUF], sems[_NBUF:]
        wid = lax.axis_index("s") * nc + lax.axis_index("c")
        base = wid * b_per_w
        pltpu.sync_copy(idx_hbm.at[pl.ds(base, b_per_w)], idx_v)

        def clamp_body(i, carry):
            off = pl.multiple_of(i * 16, 16)
            v = idx_v[pl.ds(off, 16)]
            idx_v[pl.ds(off, 16)] = jnp.clip(v, 0, V - 1)
            return carry

        lax.fori_loop(0, b_per_w // 16, clamp_body, 0)

        def start(chunk, buf):
            off = pl.multiple_of(chunk * _CHUNK, _CHUNK)
            pltpu.async_copy(
                table_hbm.at[idx_v.at[pl.ds(off, _CHUNK)]],
                rows_v.at[buf],
                gsem[buf],
            )

        def wait_gather(buf):
            pltpu.make_async_copy(
                table_hbm.at[idx_v.at[pl.ds(0, _CHUNK)]],
                rows_v.at[buf],
                gsem[buf],
            ).wait()

        def out_slice(chunk):
            return out_hbm.at[pl.ds(pl.multiple_of(base + chunk * _CHUNK, _CHUNK), _CHUNK)]

        def start_write(chunk, buf):
            pltpu.async_copy(rows_v.at[buf], out_slice(chunk), wsem[buf])

        def wait_write(buf):
            pltpu.make_async_copy(rows_v.at[buf], out_slice(0), wsem[buf]).wait()

        for b in range(_OG):
            start(b, b)

        def body(g, carry):
            for b in range(_NBUF):
                chunk = _NBUF * g + b
                nxt = chunk + _OG
                bn = (b + _OG) % _NBUF

                @pl.when(nxt < n_chunks)
                def _():
                    @pl.when(nxt >= _NBUF)
                    def _():
                        wait_write(bn)

                    start(nxt, bn)

                wait_gather(b)
                start_write(chunk, b)
            return carry

        lax.fori_loop(0, n_chunks // _NBUF, body, 0)
        for b in range(_NBUF):
            wait_write(b)

    return gather_kernel


def kernel(positions, table):
    out_shape = positions.shape + (table.shape[1],)
    flat = positions.reshape(-1)
    out = _make_gather(flat.shape[0], table.shape[0], table.shape[1])(flat, table)
    return out.reshape(out_shape)


# speedup vs baseline: 1.0063x; 1.0063x over previous
import functools

import jax
import jax.numpy as jnp
from jax import lax
from jax.experimental import pallas as pl
from jax.experimental.pallas import tpu as pltpu
from jax.experimental.pallas import tpu_sc as plsc

_MAX_LEN = 8192
_CHUNK = 8
_NBUF = 8
_OG = 5


@functools.lru_cache(maxsize=None)
def _make_gather(B, V, D):
    info = plsc.get_sparse_core_info()
    nc, ns = info.num_cores, info.num_subcores
    nw = nc * ns
    b_per_w = B // nw
    n_chunks = b_per_w // _CHUNK
    assert b_per_w * nw == B and n_chunks * _CHUNK == b_per_w
    assert n_chunks % _NBUF == 0 and n_chunks >= 2 * _NBUF

    mesh = plsc.VectorSubcoreMesh(core_axis_name="c", subcore_axis_name="s")

    @functools.partial(
        pl.kernel,
        mesh=mesh,
        out_type=jax.ShapeDtypeStruct((B, D), jnp.float32),
        scratch_types=[
            pltpu.VMEM((b_per_w,), jnp.int32),
            pltpu.VMEM((_NBUF, _CHUNK, D), jnp.float32),
        ]
        + [pltpu.SemaphoreType.DMA] * (2 * _NBUF),
    )
    def gather_kernel(idx_hbm, table_hbm, out_hbm, idx_v, rows_v, *sems):
        gsem, wsem = sems[:_NBUF], sems[_NBUF:]
        wid = lax.axis_index("s") * nc + lax.axis_index("c")
        base = wid * b_per_w
        pltpu.sync_copy(idx_hbm.at[pl.ds(base, b_per_w)], idx_v)

        def clamp_body(i, carry):
            off = pl.multiple_of(i * 16, 16)
            v = idx_v[pl.ds(off, 16)]
            idx_v[pl.ds(off, 16)] = jnp.clip(v, 0, V - 1)
            return carry

        lax.fori_loop(0, b_per_w // 16, clamp_body, 0)

        def start(chunk, buf):
            off = pl.multiple_of(chunk * _CHUNK, _CHUNK)
            pltpu.async_copy(
                table_hbm.at[idx_v.at[pl.ds(off, _CHUNK)]],
                rows_v.at[buf],
                gsem[buf],
            )

        def wait_gather(buf):
            pltpu.make_async_copy(
                table_hbm.at[idx_v.at[pl.ds(0, _CHUNK)]],
                rows_v.at[buf],
                gsem[buf],
            ).wait()

        def out_slice(chunk):
            return out_hbm.at[pl.ds(pl.multiple_of(base + chunk * _CHUNK, _CHUNK), _CHUNK)]

        def start_write(chunk, buf):
            pltpu.async_copy(rows_v.at[buf], out_slice(chunk), wsem[buf])

        def wait_write(buf):
            pltpu.make_async_copy(rows_v.at[buf], out_slice(0), wsem[buf]).wait()

        for b in range(_OG):
            start(b, b)

        def body(g, carry):
            for b in range(_NBUF):
                chunk = _NBUF * g + b
                nxt = chunk + _OG
                bn = (b + _OG) % _NBUF

                @pl.when(nxt < n_chunks)
                def _():
                    @pl.when(nxt >= _NBUF)
                    def _():
                        wait_write(bn)

                    start(nxt, bn)

                wait_gather(b)
                start_write(chunk, b)
            return carry

        lax.fori_loop(0, n_chunks // _NBUF, body, 0)
        for b in range(_NBUF):
            wait_write(b)

    return gather_kernel


def kernel(positions, table):
    out_shape = positions.shape + (table.shape[1],)
    flat = positions.reshape(-1)
    out = _make_gather(flat.shape[0], table.shape[0], table.shape[1])(flat, table)
    return out.reshape(out_shape)
